# baseline (speedup 1.0000x reference)
import jax
import jax.numpy as jnp
from jax import lax
from jax.experimental import pallas as pl
from jax.experimental.pallas import tpu as pltpu

_N = 4096
_D = 64
_C = 8
_BM = 256
_GRID = _N // _BM
_SLOTS = 8


def _slot_kernel(f1_ref, f2_ref, t1_ref, t2c_ref, t2r_ref, out_hbm, buf, sem):
    i = pl.program_id(0)
    slot = lax.rem(i, _SLOTS)

    @pl.when(i >= _SLOTS)
    def _():
        pltpu.make_async_copy(
            buf.at[slot],
            out_hbm.at[pl.ds((i - _SLOTS) * _BM, _BM)],
            sem.at[slot],
        ).wait()

    f1 = f1_ref[...]
    f2 = f2_ref[...]
    t1 = t1_ref[...]
    t2c = t2c_ref[...]
    t2r = t2r_ref[...]

    slot1 = lax.broadcasted_iota(jnp.int32, (_BM, _D), 1) // _C
    m1 = jnp.where(slot1 == t1, f1, 0.0)
    slot2 = lax.broadcasted_iota(jnp.int32, (_N, _D), 1) // _C
    m2 = jnp.where(slot2 == t2c, f2, 0.0)

    ka = lax.broadcasted_iota(jnp.int32, (_D, _D), 0) % _C
    kb = lax.broadcasted_iota(jnp.int32, (_D, _D), 1) % _C
    p = jnp.where(ka == kb, 1.0, 0.0).astype(jnp.bfloat16)

    c1 = jax.lax.dot_general(
        m1.astype(jnp.bfloat16), p, (((1,), (0,)), ((), ())),
        preferred_element_type=jnp.float32)
    cross = jax.lax.dot_general(
        c1.astype(jnp.bfloat16), m2.astype(jnp.bfloat16),
        (((1,), (1,)), ((), ())),
        preferred_element_type=jnp.float32)
    full = jax.lax.dot_general(
        f1, f2, (((1,), (1,)), ((), ())),
        preferred_element_type=jnp.float32)

    mask = t1 == t2r
    buf[slot] = jnp.where(mask, full, cross)

    pltpu.make_async_copy(
        buf.at[slot], out_hbm.at[pl.ds(i * _BM, _BM)], sem.at[slot]
    ).start()

    @pl.when(i == _GRID - 1)
    def _():
        for d in range(_SLOTS):
            j = _GRID - _SLOTS + d
            s = j % _SLOTS
            pltpu.make_async_copy(
                buf.at[s], out_hbm.at[pl.ds(j * _BM, _BM)], sem.at[s]
            ).wait()


@jax.jit
def kernel(ft_1, ft_2, type1, type2):
    t1c = type1.astype(jnp.int32).reshape(_N, 1)
    t2c = type2.astype(jnp.int32).reshape(_N, 1)
    t2r = type2.astype(jnp.int32).reshape(1, _N)

    return pl.pallas_call(
        _slot_kernel,
        grid=(_GRID,),
        in_specs=[
            pl.BlockSpec((_BM, _D), lambda i: (i, 0)),
            pl.BlockSpec((_N, _D), lambda i: (0, 0)),
            pl.BlockSpec((_BM, 1), lambda i: (i, 0)),
            pl.BlockSpec((_N, 1), lambda i: (0, 0)),
            pl.BlockSpec((1, _N), lambda i: (0, 0)),
        ],
        out_specs=pl.BlockSpec(memory_space=pltpu.MemorySpace.HBM),
        out_shape=jax.ShapeDtypeStruct((_N, _N), jnp.float32),
        scratch_shapes=[
            pltpu.VMEM((_SLOTS, _BM, _N), jnp.float32),
            pltpu.SemaphoreType.DMA((_SLOTS,)),
        ],
        compiler_params=pltpu.CompilerParams(
            vmem_limit_bytes=100 * 1024 * 1024),
    )(ft_1, ft_2, t1c, t2c, t2r)


# speedup vs baseline: 1.0593x; 1.0593x over previous
import jax
import jax.numpy as jnp
from jax import lax
from jax.experimental import pallas as pl
from jax.experimental.pallas import tpu as pltpu

_N = 4096
_D = 64
_C = 8
_BM = 512
_GRID = _N // _BM
_SLOTS = 6


def _slot_kernel(f1_ref, f2_ref, t1_ref, t2c_ref, t2r_ref, out_hbm, buf, sem):
    i = pl.program_id(0)
    slot = lax.rem(i, _SLOTS)

    @pl.when(i >= _SLOTS)
    def _():
        pltpu.make_async_copy(
            buf.at[slot],
            out_hbm.at[pl.ds((i - _SLOTS) * _BM, _BM)],
            sem.at[slot],
        ).wait()

    f1 = f1_ref[...]
    f2 = f2_ref[...]
    t1 = t1_ref[...]
    t2c = t2c_ref[...]
    t2r = t2r_ref[...]

    slot1 = lax.broadcasted_iota(jnp.int32, (_BM, _D), 1) // _C
    m1 = jnp.where(slot1 == t1, f1, 0.0)
    slot2 = lax.broadcasted_iota(jnp.int32, (_N, _D), 1) // _C
    m2 = jnp.where(slot2 == t2c, f2, 0.0)

    ka = lax.broadcasted_iota(jnp.int32, (_D, _D), 0) % _C
    kb = lax.broadcasted_iota(jnp.int32, (_D, _D), 1) % _C
    p = jnp.where(ka == kb, 1.0, 0.0).astype(jnp.bfloat16)

    c1 = jax.lax.dot_general(
        m1.astype(jnp.bfloat16), p, (((1,), (0,)), ((), ())),
        preferred_element_type=jnp.float32)
    cross = jax.lax.dot_general(
        c1.astype(jnp.bfloat16), m2.astype(jnp.bfloat16),
        (((1,), (1,)), ((), ())),
        preferred_element_type=jnp.float32)
    full = jax.lax.dot_general(
        f1, f2, (((1,), (1,)), ((), ())),
        preferred_element_type=jnp.float32)

    mask = t1 == t2r
    buf[slot] = jnp.where(mask, full, cross)

    pltpu.make_async_copy(
        buf.at[slot], out_hbm.at[pl.ds(i * _BM, _BM)], sem.at[slot]
    ).start()

    @pl.when(i == _GRID - 1)
    def _():
        for d in range(_SLOTS):
            j = _GRID - _SLOTS + d
            s = j % _SLOTS
            pltpu.make_async_copy(
                buf.at[s], out_hbm.at[pl.ds(j * _BM, _BM)], sem.at[s]
            ).wait()


@jax.jit
def kernel(ft_1, ft_2, type1, type2):
    t1c = type1.astype(jnp.int32).reshape(_N, 1)
    t2c = type2.astype(jnp.int32).reshape(_N, 1)
    t2r = type2.astype(jnp.int32).reshape(1, _N)

    return pl.pallas_call(
        _slot_kernel,
        grid=(_GRID,),
        in_specs=[
            pl.BlockSpec((_BM, _D), lambda i: (i, 0)),
            pl.BlockSpec((_N, _D), lambda i: (0, 0)),
            pl.BlockSpec((_BM, 1), lambda i: (i, 0)),
            pl.BlockSpec((_N, 1), lambda i: (0, 0)),
            pl.BlockSpec((1, _N), lambda i: (0, 0)),
        ],
        out_specs=pl.BlockSpec(memory_space=pltpu.MemorySpace.HBM),
        out_shape=jax.ShapeDtypeStruct((_N, _N), jnp.float32),
        scratch_shapes=[
            pltpu.VMEM((_SLOTS, _BM, _N), jnp.float32),
            pltpu.SemaphoreType.DMA((_SLOTS,)),
        ],
        compiler_params=pltpu.CompilerParams(
            vmem_limit_bytes=100 * 1024 * 1024),
    )(ft_1, ft_2, t1c, t2c, t2r)


# speedup vs baseline: 1.0610x; 1.0015x over previous
import jax
import jax.numpy as jnp
from jax import lax
from jax.experimental import pallas as pl
from jax.experimental.pallas import tpu as pltpu

_N = 4096
_D = 64
_C = 8
_BM = 512
_GRID = _N // _BM
_SLOTS = 4


def _slot_kernel(f1_ref, f2_ref, t1_ref, t2c_ref, t2r_ref, out_hbm, buf, sem):
    i = pl.program_id(0)
    slot = lax.rem(i, _SLOTS)

    @pl.when(i >= _SLOTS)
    def _():
        pltpu.make_async_copy(
            buf.at[slot],
            out_hbm.at[pl.ds((i - _SLOTS) * _BM, _BM)],
            sem.at[slot],
        ).wait()

    f1 = f1_ref[...]
    f2 = f2_ref[...]
    t1 = t1_ref[...]
    t2c = t2c_ref[...]
    t2r = t2r_ref[...]

    slot1 = lax.broadcasted_iota(jnp.int32, (_BM, _D), 1) // _C
    m1 = jnp.where(slot1 == t1, f1, 0.0)
    slot2 = lax.broadcasted_iota(jnp.int32, (_N, _D), 1) // _C
    m2 = jnp.where(slot2 == t2c, f2, 0.0)

    ka = lax.broadcasted_iota(jnp.int32, (_D, _D), 0) % _C
    kb = lax.broadcasted_iota(jnp.int32, (_D, _D), 1) % _C
    p = jnp.where(ka == kb, 1.0, 0.0).astype(jnp.bfloat16)

    c1 = jax.lax.dot_general(
        m1.astype(jnp.bfloat16), p, (((1,), (0,)), ((), ())),
        preferred_element_type=jnp.float32)
    cross = jax.lax.dot_general(
        c1.astype(jnp.bfloat16), m2.astype(jnp.bfloat16),
        (((1,), (1,)), ((), ())),
        preferred_element_type=jnp.float32)
    full = jax.lax.dot_general(
        f1, f2, (((1,), (1,)), ((), ())),
        preferred_element_type=jnp.float32)

    mask = t1 == t2r
    buf[slot] = jnp.where(mask, full, cross)

    pltpu.make_async_copy(
        buf.at[slot], out_hbm.at[pl.ds(i * _BM, _BM)], sem.at[slot]
    ).start()

    @pl.when(i == _GRID - 1)
    def _():
        for d in range(_SLOTS):
            j = _GRID - _SLOTS + d
            s = j % _SLOTS
            pltpu.make_async_copy(
                buf.at[s], out_hbm.at[pl.ds(j * _BM, _BM)], sem.at[s]
            ).wait()


@jax.jit
def kernel(ft_1, ft_2, type1, type2):
    t1c = type1.astype(jnp.int32).reshape(_N, 1)
    t2c = type2.astype(jnp.int32).reshape(_N, 1)
    t2r = type2.astype(jnp.int32).reshape(1, _N)

    return pl.pallas_call(
        _slot_kernel,
        grid=(_GRID,),
        in_specs=[
            pl.BlockSpec((_BM, _D), lambda i: (i, 0)),
            pl.BlockSpec((_N, _D), lambda i: (0, 0)),
            pl.BlockSpec((_BM, 1), lambda i: (i, 0)),
            pl.BlockSpec((_N, 1), lambda i: (0, 0)),
            pl.BlockSpec((1, _N), lambda i: (0, 0)),
        ],
        out_specs=pl.BlockSpec(memory_space=pltpu.MemorySpace.HBM),
        out_shape=jax.ShapeDtypeStruct((_N, _N), jnp.float32),
        scratch_shapes=[
            pltpu.VMEM((_SLOTS, _BM, _N), jnp.float32),
            pltpu.SemaphoreType.DMA((_SLOTS,)),
        ],
        compiler_params=pltpu.CompilerParams(
            vmem_limit_bytes=100 * 1024 * 1024),
    )(ft_1, ft_2, t1c, t2c, t2r)
